# Initial kernel scaffold; baseline (speedup 1.0000x reference)
import jax
import jax.numpy as jnp
from jax.experimental import pallas as pl


def kernel(x, edge_index, edge_attr, edge_weight, W_neigh, b_neigh, W_root, b_root, edge_emb):
    raise NotImplementedError("write your pallas kernel here")


# speedup vs baseline: 4.3423x; 4.3423x over previous
import functools

import jax
import jax.numpy as jnp
from jax.experimental import pallas as pl
from jax.experimental.pallas import tpu as pltpu
from jax.experimental.pallas import tpu_sc as plsc

_NC = 2
_NS = 16
_LANES = 16
_B = 80
_CHUNK = 4000


def _matmul_call(x, Wcat, bcat, blk):
    N, D = x.shape
    D2 = Wcat.shape[1]
    H = D // 2

    def body(x_ref, w_ref, b_ref, neigh_ref, root_ref):
        z = jnp.dot(x_ref[...], w_ref[...], preferred_element_type=jnp.float32)
        z = z + b_ref[...]
        neigh_ref[0] = z[:, :H]
        neigh_ref[1] = z[:, H:D]
        root_ref[...] = z[:, D:]

    return pl.pallas_call(
        body,
        grid=(N // blk,),
        in_specs=[
            pl.BlockSpec((blk, D), lambda i: (i, 0)),
            pl.BlockSpec((D, D2), lambda i: (0, 0)),
            pl.BlockSpec((1, D2), lambda i: (0, 0)),
        ],
        out_specs=[
            pl.BlockSpec((2, blk, H), lambda i: (0, i, 0)),
            pl.BlockSpec((blk, D), lambda i: (i, 0)),
        ],
        out_shape=[
            jax.ShapeDtypeStruct((2, N, H), jnp.float32),
            jax.ShapeDtypeStruct((N, D), jnp.float32),
        ],
    )(x, Wcat, bcat)


def _sc_call(neigh2, src, dst, attr, w):
    N2, H = neigh2.shape
    N = N2 // 2
    E = src.shape[0]
    ept = E // _NS
    nchunk = ept // _CHUNK
    nblk = _CHUNK // _B
    ctiles = 10
    rpt = N // ctiles
    zrows = 125
    mesh = plsc.VectorSubcoreMesh(core_axis_name="c", subcore_axis_name="s")

    @functools.partial(
        pl.kernel,
        out_type=[
            jax.ShapeDtypeStruct((_NC, N, H), jnp.float32),
            jax.ShapeDtypeStruct((N, _LANES), jnp.float32),
        ],
        mesh=mesh,
        scratch_types=[
            pltpu.VMEM((_CHUNK,), jnp.int32),
            pltpu.VMEM((_CHUNK,), jnp.int32),
            pltpu.VMEM((_CHUNK,), jnp.int32),
            pltpu.VMEM((_CHUNK,), jnp.float32),
            pltpu.VMEM((_B, H), jnp.float32),
            pltpu.VMEM((_B, _LANES), jnp.float32),
            pltpu.VMEM((_B,), jnp.int32),
            pltpu.VMEM((zrows, H), jnp.float32),
            pltpu.VMEM((rpt, _LANES), jnp.float32),
            pltpu.VMEM_SHARED((N, H), jnp.float32),
            pltpu.VMEM_SHARED((N, _LANES), jnp.float32),
        ],
        compiler_params=pltpu.CompilerParams(needs_layout_passes=False,
                                             use_tc_tiling_on_sc=False),
    )
    def sc_kernel(neigh_hbm, src_hbm, dst_hbm, attr_hbm, w_hbm,
                  outf_hbm, outm_hbm,
                  src_c, dst_c, attr_c, w_c, rows, contrib,
                  dst_blk, zbuf, zbuf_m, accf_sh, accm_sh):
        cid = jax.lax.axis_index("c")
        sid = jax.lax.axis_index("s")
        z16 = jnp.zeros((_LANES,), jnp.float32)
        is_c0 = cid == 0
        row0 = sid * rpt

        @pl.when(sid < ctiles)
        def _():
            @pl.loop(0, zrows)
            def _(i):
                for j in range(H // _LANES):
                    zbuf[i, pl.ds(j * _LANES, _LANES)] = z16

            for k in range(rpt // zrows):
                pltpu.sync_copy(zbuf,
                                accf_sh.at[pl.ds(row0 + k * zrows, zrows)])

        @pl.when((sid < ctiles) & is_c0)
        def _():
            @pl.loop(0, rpt)
            def _(i):
                zbuf_m[i] = z16

            pltpu.sync_copy(zbuf_m, accm_sh.at[pl.ds(row0, rpt)])

        plsc.subcore_barrier()

        ebase = sid * ept
        iota = jax.lax.broadcasted_iota(jnp.int32, (_LANES,), 0)
        coff = cid * N

        @pl.loop(0, nchunk)
        def _(c):
            cb = ebase + c * _CHUNK
            pltpu.sync_copy(src_hbm.at[pl.ds(cb, _CHUNK)], src_c)
            pltpu.sync_copy(dst_hbm.at[pl.ds(cb, _CHUNK)], dst_c)
            pltpu.sync_copy(w_hbm.at[pl.ds(cb, _CHUNK)], w_c)

            @pl.when(is_c0)
            def _():
                pltpu.sync_copy(attr_hbm.at[pl.ds(cb, _CHUNK)], attr_c)

            @pl.loop(0, _CHUNK // _LANES)
            def _(k):
                sl = pl.ds(k * _LANES, _LANES)
                src_c[sl] = src_c[sl] + coff

            @pl.loop(0, nblk)
            def _(b):
                off = b * _B
                for k in range(_B // _LANES):
                    dst_blk[pl.ds(k * _LANES, _LANES)] = (
                        dst_c[pl.ds(off + k * _LANES, _LANES)])
                pltpu.sync_copy(neigh_hbm.at[src_c.at[pl.ds(off, _B)]], rows)

                @pl.loop(0, _B)
                def _(e):
                    esplat = jnp.zeros((_LANES,), jnp.int32) + (off + e)
                    wv = plsc.load_gather(w_c, [esplat])
                    for j in range(H // _LANES):
                        sl = pl.ds(j * _LANES, _LANES)
                        rows[e, sl] = rows[e, sl] * wv

                    @pl.when(is_c0)
                    def _():
                        av = plsc.load_gather(attr_c, [esplat])
                        contrib[e] = (jnp.where(iota == 0, 1.0, 0.0)
                                      + jnp.where(iota == av + 1, wv, 0.0))

                pltpu.sync_copy(rows, accf_sh.at[dst_blk], add=True)

                @pl.when(is_c0)
                def _():
                    pltpu.sync_copy(contrib, accm_sh.at[dst_blk], add=True)

        plsc.subcore_barrier()

        @pl.when(sid < ctiles)
        def _():
            pltpu.sync_copy(accf_sh.at[pl.ds(row0, rpt)],
                            outf_hbm.at[cid, pl.ds(row0, rpt)])

        @pl.when((sid < ctiles) & is_c0)
        def _():
            pltpu.sync_copy(accm_sh.at[pl.ds(row0, rpt)],
                            outm_hbm.at[pl.ds(row0, rpt)])

    return sc_kernel(neigh2, src, dst, attr, w)


def _combine_call(accf, accm, root, P, e0, blk):
    _, N, H = accf.shape
    D = 2 * H

    def body(accf_ref, accm_ref, root_ref, p_ref, e0_ref, out_ref):
        feat = jnp.concatenate([accf_ref[0], accf_ref[1]], axis=1)
        meta = accm_ref[...]
        term2 = jnp.dot(meta, p_ref[...], preferred_element_type=jnp.float32)
        cnt = jnp.dot(meta, e0_ref[...], preferred_element_type=jnp.float32)
        agg = (feat + term2) / jnp.maximum(cnt, 1.0)
        o = agg + root_ref[...]
        out_ref[...] = jnp.where(o >= 0, o, 0.01 * o)

    return pl.pallas_call(
        body,
        grid=(N // blk,),
        in_specs=[
            pl.BlockSpec((2, blk, H), lambda i: (0, i, 0)),
            pl.BlockSpec((blk, _LANES), lambda i: (i, 0)),
            pl.BlockSpec((blk, D), lambda i: (i, 0)),
            pl.BlockSpec((_LANES, D), lambda i: (0, 0)),
            pl.BlockSpec((_LANES, 1), lambda i: (0, 0)),
        ],
        out_specs=pl.BlockSpec((blk, D), lambda i: (i, 0)),
        out_shape=jax.ShapeDtypeStruct((N, D), jnp.float32),
    )(accf, accm, root, P, e0)


def kernel(x, edge_index, edge_attr, edge_weight,
           W_neigh, b_neigh, W_root, b_root, edge_emb):
    N, D = x.shape
    src = edge_index[0].astype(jnp.int32)
    dst = edge_index[1].astype(jnp.int32)
    attr = edge_attr[:, 0].astype(jnp.int32)
    w = edge_weight[:, 0].astype(jnp.float32)

    Wcat = jnp.concatenate([W_neigh.T, W_root.T], axis=1)
    bcat = jnp.concatenate([b_neigh, b_root])[None, :]
    neigh2, root = _matmul_call(x, Wcat, bcat, blk=1000)

    accf, accm = _sc_call(neigh2.reshape(2 * N, D // 2), src, dst, attr, w)

    nb = edge_emb.shape[0]
    P = jnp.zeros((_LANES, D), jnp.float32).at[1:1 + nb].set(edge_emb)
    e0 = jnp.zeros((_LANES, 1), jnp.float32).at[0, 0].set(1.0)
    return _combine_call(accf, accm, root, P, e0, blk=1000)


# speedup vs baseline: 8.5395x; 1.9666x over previous
import functools

import jax
import jax.numpy as jnp
from jax.experimental import pallas as pl
from jax.experimental.pallas import tpu as pltpu
from jax.experimental.pallas import tpu_sc as plsc

_NC = 2
_NS = 16
_LANES = 16
_B = 80
_CHUNK = 4000
_NMETA = 5


def _matmul_call(x, Wcat, bcat, blk):
    N, D = x.shape
    D2 = Wcat.shape[1]
    H = D // 2

    def body(x_ref, w_ref, b_ref, neigh_ref, root_ref):
        z = jnp.dot(x_ref[...], w_ref[...], preferred_element_type=jnp.float32)
        z = z + b_ref[...]
        neigh_ref[0] = z[:, :H]
        neigh_ref[1] = z[:, H:D]
        root_ref[...] = z[:, D:]

    return pl.pallas_call(
        body,
        grid=(N // blk,),
        in_specs=[
            pl.BlockSpec((blk, D), lambda i: (i, 0)),
            pl.BlockSpec((D, D2), lambda i: (0, 0)),
            pl.BlockSpec((1, D2), lambda i: (0, 0)),
        ],
        out_specs=[
            pl.BlockSpec((2, blk, H), lambda i: (0, i, 0)),
            pl.BlockSpec((blk, D), lambda i: (i, 0)),
        ],
        out_shape=[
            jax.ShapeDtypeStruct((2, N, H), jnp.float32),
            jax.ShapeDtypeStruct((N, D), jnp.float32),
        ],
    )(x, Wcat, bcat)


def _sc_call(neigh2, src, dst, attr, w):
    N2, H = neigh2.shape
    N = N2 // 2
    E = src.shape[0]
    ept = E // _NS
    nchunk = ept // _CHUNK
    nblk = _CHUNK // _B
    ctiles = 10
    rpt = N // ctiles
    zrows = 125
    mesh = plsc.VectorSubcoreMesh(core_axis_name="c", subcore_axis_name="s")

    @functools.partial(
        pl.kernel,
        out_type=[
            jax.ShapeDtypeStruct((_NC, N, H), jnp.float32),
            jax.ShapeDtypeStruct((_NC, _NS, _NMETA * N), jnp.float32),
        ],
        mesh=mesh,
        scratch_types=[
            pltpu.VMEM((_CHUNK,), jnp.int32),
            pltpu.VMEM((_CHUNK,), jnp.int32),
            pltpu.VMEM((_CHUNK,), jnp.int32),
            pltpu.VMEM((_CHUNK,), jnp.float32),
            pltpu.VMEM((_B, H), jnp.float32),
            pltpu.VMEM((_B, H), jnp.float32),
            pltpu.VMEM((_B,), jnp.int32),
            pltpu.VMEM((_B,), jnp.int32),
            pltpu.VMEM((zrows, H), jnp.float32),
            pltpu.VMEM((_NMETA * N,), jnp.float32),
            pltpu.VMEM_SHARED((N, H), jnp.float32),
            pltpu.SemaphoreType.DMA,
            pltpu.SemaphoreType.DMA,
            pltpu.SemaphoreType.DMA,
            pltpu.SemaphoreType.DMA,
        ],
        compiler_params=pltpu.CompilerParams(needs_layout_passes=False,
                                             use_tc_tiling_on_sc=False),
    )
    def sc_kernel(neigh_hbm, src_hbm, dst_hbm, attr_hbm, w_hbm,
                  outf_hbm, outm_hbm,
                  src_c, dst_c, attr_c, w_c,
                  rows0, rows1, dstb0, dstb1, zbuf, meta_t, accf_sh,
                  gsem0, gsem1, fsem0, fsem1):
        cid = jax.lax.axis_index("c")
        sid = jax.lax.axis_index("s")
        z16 = jnp.zeros((_LANES,), jnp.float32)
        ones16 = jnp.ones((_LANES,), jnp.float32)
        row0 = sid * rpt
        rows_b = (rows0, rows1)
        dstb_b = (dstb0, dstb1)
        gsem_b = (gsem0, gsem1)
        fsem_b = (fsem0, fsem1)

        @pl.when(sid < ctiles)
        def _():
            @pl.loop(0, zrows)
            def _(i):
                for j in range(H // _LANES):
                    zbuf[i, pl.ds(j * _LANES, _LANES)] = z16

            for k in range(rpt // zrows):
                pltpu.sync_copy(zbuf,
                                accf_sh.at[pl.ds(row0 + k * zrows, zrows)])

        @pl.loop(0, _NMETA * N // _LANES, unroll=8)
        def _(i):
            meta_t[pl.ds(i * _LANES, _LANES)] = z16

        plsc.subcore_barrier()

        ebase = sid * ept
        coff = cid * N

        def gather_start(g, p):
            pltpu.async_copy(neigh_hbm.at[src_c.at[pl.ds(g * _B, _B)]],
                             rows_b[p], gsem_b[p])

        def gather_wait(p):
            pltpu.make_async_copy(neigh_hbm.at[src_c.at[pl.ds(0, _B)]],
                                  rows_b[p], gsem_b[p]).wait()

        def compute(g, p):
            rows, dstb = rows_b[p], dstb_b[p]
            off = g * _B
            for k in range(_B // _LANES):
                dstb[pl.ds(k * _LANES, _LANES)] = (
                    dst_c[pl.ds(off + k * _LANES, _LANES)])

            @pl.loop(0, _B, unroll=4)
            def _(e):
                esplat = jnp.zeros((_LANES,), jnp.int32) + (off + e)
                wv = plsc.load_gather(w_c, [esplat])
                for j in range(H // _LANES):
                    sl = pl.ds(j * _LANES, _LANES)
                    rows[e, sl] = rows[e, sl] * wv

            for k in range(_B // _LANES):
                sl = pl.ds(off + k * _LANES, _LANES)
                d16 = dst_c[sl]
                a16 = attr_c[sl]
                w16 = w_c[sl]
                plsc.addupdate_scatter(meta_t, [d16], ones16)
                plsc.addupdate_scatter(meta_t, [(a16 + 1) * N + d16], w16)

        def scatter_start(p):
            pltpu.async_copy(rows_b[p], accf_sh.at[dstb_b[p]], fsem_b[p],
                             add=True)

        def scatter_wait(p):
            pltpu.make_async_copy(rows_b[p], accf_sh.at[dstb_b[p]],
                                  fsem_b[p]).wait()

        @pl.loop(0, nchunk)
        def _(c):
            cb = ebase + c * _CHUNK
            pltpu.sync_copy(src_hbm.at[pl.ds(cb, _CHUNK)], src_c)
            pltpu.sync_copy(dst_hbm.at[pl.ds(cb, _CHUNK)], dst_c)
            pltpu.sync_copy(attr_hbm.at[pl.ds(cb, _CHUNK)], attr_c)
            pltpu.sync_copy(w_hbm.at[pl.ds(cb, _CHUNK)], w_c)

            @pl.loop(0, _CHUNK // _LANES, unroll=8)
            def _(k):
                sl = pl.ds(k * _LANES, _LANES)
                src_c[sl] = src_c[sl] + coff

            gather_start(0, 0)

            @pl.loop(0, nblk // 2)
            def _(h):
                for p in range(2):
                    g = h * 2 + p
                    gather_wait(p)

                    if p == 1:
                        scatter_wait(0)

                        @pl.when(h < nblk // 2 - 1)
                        def _():
                            gather_start(g + 1, 0)
                    else:
                        @pl.when(h >= 1)
                        def _():
                            scatter_wait(1)

                        gather_start(g + 1, 1)

                    compute(g, p)
                    scatter_start(p)

            scatter_wait(1)

        plsc.subcore_barrier()

        pltpu.sync_copy(meta_t, outm_hbm.at[cid, sid])

        @pl.when(sid < ctiles)
        def _():
            pltpu.sync_copy(accf_sh.at[pl.ds(row0, rpt)],
                            outf_hbm.at[cid, pl.ds(row0, rpt)])

    return sc_kernel(neigh2, src, dst, attr, w)


def _meta_call(accm, P5, e5):
    _, _, _, N = accm.shape
    D = P5.shape[1]
    dn = (((0,), (0,)), ((), ()))

    def body(m_ref, p_ref, e5_ref, t2_ref, cnt_ref):
        meta = jnp.sum(m_ref[...], axis=(0, 1)) * 0.5
        t2_ref[...] = jax.lax.dot_general(
            meta, p_ref[...], dn, preferred_element_type=jnp.float32)
        cnt_ref[...] = jax.lax.dot_general(
            meta, e5_ref[...], dn, preferred_element_type=jnp.float32)

    return pl.pallas_call(
        body,
        out_shape=[
            jax.ShapeDtypeStruct((N, D), jnp.float32),
            jax.ShapeDtypeStruct((N, 1), jnp.float32),
        ],
    )(accm, P5, e5)


def _combine_call(accf, term2, cnt, root, blk):
    _, N, H = accf.shape
    D = 2 * H

    def body(accf_ref, t2_ref, cnt_ref, root_ref, out_ref):
        feat = jnp.concatenate([accf_ref[0], accf_ref[1]], axis=1)
        agg = (feat + t2_ref[...]) / jnp.maximum(cnt_ref[...], 1.0)
        o = agg + root_ref[...]
        out_ref[...] = jnp.where(o >= 0, o, 0.01 * o)

    return pl.pallas_call(
        body,
        grid=(N // blk,),
        in_specs=[
            pl.BlockSpec((2, blk, H), lambda i: (0, i, 0)),
            pl.BlockSpec((blk, D), lambda i: (i, 0)),
            pl.BlockSpec((blk, 1), lambda i: (i, 0)),
            pl.BlockSpec((blk, D), lambda i: (i, 0)),
        ],
        out_specs=pl.BlockSpec((blk, D), lambda i: (i, 0)),
        out_shape=jax.ShapeDtypeStruct((N, D), jnp.float32),
    )(accf, term2, cnt, root)


def kernel(x, edge_index, edge_attr, edge_weight,
           W_neigh, b_neigh, W_root, b_root, edge_emb):
    N, D = x.shape
    src = edge_index[0].astype(jnp.int32)
    dst = edge_index[1].astype(jnp.int32)
    attr = edge_attr[:, 0].astype(jnp.int32)
    w = edge_weight[:, 0].astype(jnp.float32)

    Wcat = jnp.concatenate([W_neigh.T, W_root.T], axis=1)
    bcat = jnp.concatenate([b_neigh, b_root])[None, :]
    neigh2, root = _matmul_call(x, Wcat, bcat, blk=1000)

    accf, accm = _sc_call(neigh2.reshape(2 * N, D // 2), src, dst, attr, w)
    accm = accm.reshape(_NC, _NS, _NMETA, N)

    nb = edge_emb.shape[0]
    P5 = jnp.zeros((_NMETA, D), jnp.float32).at[1:1 + nb].set(edge_emb)
    e5 = jnp.zeros((_NMETA, 1), jnp.float32).at[0, 0].set(1.0)
    term2, cnt = _meta_call(accm, P5, e5)
    return _combine_call(accf, term2, cnt, root, blk=1000)


# speedup vs baseline: 9.2023x; 1.0776x over previous
import functools

import jax
import jax.numpy as jnp
from jax.experimental import pallas as pl
from jax.experimental.pallas import tpu as pltpu
from jax.experimental.pallas import tpu_sc as plsc

_NC = 2
_NS = 16
_LANES = 16
_B = 80
_CHUNK = 4000
_NMETA = 5
_DNT = (((1,), (1,)), ((), ()))
_DN0 = (((0,), (0,)), ((), ()))


def _matmul_call(x, W_neigh, b_neigh, W_root, b_root, blk):
    N, D = x.shape

    def body(x_ref, wn_ref, bn_ref, wr_ref, br_ref, neigh_ref, root_ref):
        xb = x_ref[...]
        neigh_ref[...] = jax.lax.dot_general(
            xb, wn_ref[...], _DNT,
            preferred_element_type=jnp.float32) + bn_ref[...]
        root_ref[...] = jax.lax.dot_general(
            xb, wr_ref[...], _DNT,
            preferred_element_type=jnp.float32) + br_ref[...]

    return pl.pallas_call(
        body,
        grid=(N // blk,),
        in_specs=[
            pl.BlockSpec((blk, D), lambda i: (i, 0)),
            pl.BlockSpec((D, D), lambda i: (0, 0)),
            pl.BlockSpec((1, D), lambda i: (0, 0)),
            pl.BlockSpec((D, D), lambda i: (0, 0)),
            pl.BlockSpec((1, D), lambda i: (0, 0)),
        ],
        out_specs=[
            pl.BlockSpec((blk, D), lambda i: (i, 0)),
            pl.BlockSpec((blk, D), lambda i: (i, 0)),
        ],
        out_shape=[
            jax.ShapeDtypeStruct((N, D), jnp.float32),
            jax.ShapeDtypeStruct((N, D), jnp.float32),
        ],
    )(x, W_neigh, b_neigh, W_root, b_root)


def _sc_call(neigh2, src, dst, attr, w):
    N2, H = neigh2.shape
    N = N2 // 2
    E = src.shape[0]
    ept = E // _NS
    nchunk = ept // _CHUNK
    nblk = _CHUNK // _B
    ctiles = 10
    rpt = N // ctiles
    zrows = 125
    mesh = plsc.VectorSubcoreMesh(core_axis_name="c", subcore_axis_name="s")

    @functools.partial(
        pl.kernel,
        out_type=[
            jax.ShapeDtypeStruct((_NC, N, H), jnp.float32),
            jax.ShapeDtypeStruct((_NS, _NMETA * N), jnp.float32),
        ],
        mesh=mesh,
        scratch_types=[
            pltpu.VMEM((_CHUNK,), jnp.int32),
            pltpu.VMEM((_CHUNK,), jnp.int32),
            pltpu.VMEM((_CHUNK,), jnp.int32),
            pltpu.VMEM((_CHUNK,), jnp.float32),
            pltpu.VMEM((_B, H), jnp.float32),
            pltpu.VMEM((_B, H), jnp.float32),
            pltpu.VMEM((_B,), jnp.int32),
            pltpu.VMEM((_B,), jnp.int32),
            pltpu.VMEM((zrows, H), jnp.float32),
            pltpu.VMEM((_NMETA * N,), jnp.float32),
            pltpu.VMEM_SHARED((N, H), jnp.float32),
            pltpu.SemaphoreType.DMA,
            pltpu.SemaphoreType.DMA,
            pltpu.SemaphoreType.DMA,
            pltpu.SemaphoreType.DMA,
        ],
        compiler_params=pltpu.CompilerParams(needs_layout_passes=False,
                                             use_tc_tiling_on_sc=False),
    )
    def sc_kernel(neigh_hbm, src_hbm, dst_hbm, attr_hbm, w_hbm,
                  outf_hbm, outm_hbm,
                  src_c, dst_c, attr_c, w_c,
                  rows0, rows1, dstb0, dstb1, zbuf, meta_t, accf_sh,
                  gsem0, gsem1, fsem0, fsem1):
        cid = jax.lax.axis_index("c")
        sid = jax.lax.axis_index("s")
        is_c0 = cid == 0
        z16 = jnp.zeros((_LANES,), jnp.float32)
        ones16 = jnp.ones((_LANES,), jnp.float32)
        row0 = sid * rpt
        rows_b = (rows0, rows1)
        dstb_b = (dstb0, dstb1)
        gsem_b = (gsem0, gsem1)
        fsem_b = (fsem0, fsem1)

        @pl.when(sid < ctiles)
        def _():
            @pl.loop(0, zrows)
            def _(i):
                for j in range(H // _LANES):
                    zbuf[i, pl.ds(j * _LANES, _LANES)] = z16

            for k in range(rpt // zrows):
                pltpu.sync_copy(zbuf,
                                accf_sh.at[pl.ds(row0 + k * zrows, zrows)])

        @pl.when(is_c0)
        def _():
            @pl.loop(0, _NMETA * N // _LANES, unroll=8)
            def _(i):
                meta_t[pl.ds(i * _LANES, _LANES)] = z16

        plsc.subcore_barrier()

        ebase = sid * ept

        def gather_start(g, p):
            pltpu.async_copy(neigh_hbm.at[src_c.at[pl.ds(g * _B, _B)]],
                             rows_b[p], gsem_b[p])

        def gather_wait(p):
            pltpu.make_async_copy(neigh_hbm.at[src_c.at[pl.ds(0, _B)]],
                                  rows_b[p], gsem_b[p]).wait()

        def compute(g, p):
            rows, dstb = rows_b[p], dstb_b[p]
            off = g * _B
            for k in range(_B // _LANES):
                dstb[pl.ds(k * _LANES, _LANES)] = (
                    dst_c[pl.ds(off + k * _LANES, _LANES)])

            @pl.loop(0, _B, unroll=8)
            def _(e):
                esplat = jnp.zeros((_LANES,), jnp.int32) + (off + e)
                wv = plsc.load_gather(w_c, [esplat])
                for j in range(H // _LANES):
                    sl = pl.ds(j * _LANES, _LANES)
                    rows[e, sl] = rows[e, sl] * wv

            @pl.when(is_c0)
            def _():
                for k in range(_B // _LANES):
                    sl = pl.ds(off + k * _LANES, _LANES)
                    d16 = dst_c[sl]
                    a16 = attr_c[sl]
                    w16 = w_c[sl]
                    plsc.addupdate_scatter(meta_t, [d16], ones16)
                    plsc.addupdate_scatter(meta_t, [(a16 + 1) * N + d16], w16)

        def scatter_start(p):
            pltpu.async_copy(rows_b[p], accf_sh.at[dstb_b[p]], fsem_b[p],
                             add=True)

        def scatter_wait(p):
            pltpu.make_async_copy(rows_b[p], accf_sh.at[dstb_b[p]],
                                  fsem_b[p]).wait()

        @pl.loop(0, nchunk)
        def _(c):
            cb = ebase + c * _CHUNK
            pltpu.sync_copy(src_hbm.at[pl.ds(cb, _CHUNK)], src_c)
            pltpu.sync_copy(dst_hbm.at[pl.ds(cb, _CHUNK)], dst_c)
            pltpu.sync_copy(attr_hbm.at[pl.ds(cb, _CHUNK)], attr_c)
            pltpu.sync_copy(w_hbm.at[pl.ds(cb, _CHUNK)], w_c)

            @pl.loop(0, _CHUNK // _LANES, unroll=8)
            def _(k):
                sl = pl.ds(k * _LANES, _LANES)
                src_c[sl] = src_c[sl] * 2 + cid

            gather_start(0, 0)

            @pl.loop(0, nblk // 2)
            def _(h):
                for p in range(2):
                    g = h * 2 + p
                    gather_wait(p)

                    if p == 1:
                        scatter_wait(0)

                        @pl.when(h < nblk // 2 - 1)
                        def _():
                            gather_start(g + 1, 0)
                    else:
                        @pl.when(h >= 1)
                        def _():
                            scatter_wait(1)

                        gather_start(g + 1, 1)

                    compute(g, p)
                    scatter_start(p)

            scatter_wait(1)

        plsc.subcore_barrier()

        @pl.when(is_c0)
        def _():
            pltpu.sync_copy(meta_t, outm_hbm.at[sid])

        @pl.when(sid < ctiles)
        def _():
            pltpu.sync_copy(accf_sh.at[pl.ds(row0, rpt)],
                            outf_hbm.at[cid, pl.ds(row0, rpt)])

    return sc_kernel(neigh2, src, dst, attr, w)


def _combine_call(accf, accm, root, P5, e5):
    _, N, H = accf.shape
    D = 2 * H

    def body(accf_ref, accm_ref, root_ref, p_ref, e5_ref, out_ref):
        feat = jnp.concatenate([accf_ref[0], accf_ref[1]], axis=1)
        meta = jnp.sum(accm_ref[...], axis=0)
        term2 = jax.lax.dot_general(meta, p_ref[...], _DN0,
                                    preferred_element_type=jnp.float32)
        cnt = jax.lax.dot_general(meta, e5_ref[...], _DN0,
                                  preferred_element_type=jnp.float32)
        agg = (feat + term2) / jnp.maximum(cnt, 1.0)
        o = agg + root_ref[...]
        out_ref[...] = jnp.where(o >= 0, o, 0.01 * o)

    return pl.pallas_call(
        body,
        out_shape=jax.ShapeDtypeStruct((N, D), jnp.float32),
    )(accf, accm, root, P5, e5)


def kernel(x, edge_index, edge_attr, edge_weight,
           W_neigh, b_neigh, W_root, b_root, edge_emb):
    N, D = x.shape
    E = edge_index.shape[1]
    src = edge_index[0].astype(jnp.int32)
    dst = edge_index[1].astype(jnp.int32)
    attr = edge_attr.reshape(E).astype(jnp.int32)
    w = edge_weight.reshape(E).astype(jnp.float32)

    neigh_x, root = _matmul_call(x, W_neigh, b_neigh[None], W_root,
                                 b_root[None], blk=1000)
    neigh2 = neigh_x.reshape(2 * N, D // 2)

    accf, accm = _sc_call(neigh2, src, dst, attr, w)
    accm = accm.reshape(_NS, _NMETA, N)

    nb = edge_emb.shape[0]
    P5 = jnp.zeros((_NMETA, D), jnp.float32).at[1:1 + nb].set(edge_emb)
    e5 = jnp.zeros((_NMETA, 1), jnp.float32).at[0, 0].set(1.0)
    return _combine_call(accf, accm, root, P5, e5)
